# Initial kernel scaffold; baseline (speedup 1.0000x reference)
import jax
import jax.numpy as jnp
from jax.experimental import pallas as pl


def kernel(logits):
    raise NotImplementedError("write your pallas kernel here")


# speedup vs baseline: 3.4529x; 3.4529x over previous
import jax
import jax.numpy as jnp
from jax.experimental import pallas as pl
from jax.experimental.pallas import tpu as pltpu

_K = 64
_W = 250


def _row_kernel(x_ref, o_ref, cand_ref):
    _NEG = jnp.float32(-jnp.inf)
    _BIG = jnp.int32(2**30)
    x = x_ref[0]
    C = x.shape[0]
    mr = 125 if C % 125 == 0 else 128
    R = C // mr

    cmax = jnp.max(x, axis=1).reshape(R, mr)
    chunk_iota = (jax.lax.broadcasted_iota(jnp.int32, (R, mr), 0) * mr
                  + jax.lax.broadcasted_iota(jnp.int32, (R, mr), 1))

    row64_iota = jax.lax.broadcasted_iota(jnp.int32, (_K, 1), 0)

    def sel_body(k, st):
        m_vals, cbase = st
        mx = jnp.max(m_vals)
        pos = jnp.min(jnp.where(m_vals == mx, chunk_iota, _BIG))
        cand_ref[pl.ds(k, 1), :] = x_ref[0, pl.ds(pos, 1), :]
        cbase = jnp.where(row64_iota == k, pos, cbase)
        m_vals = jnp.where(chunk_iota == pos, _NEG, m_vals)
        return m_vals, cbase

    cbase0 = jnp.zeros((_K, 1), jnp.int32)
    _, cbase = jax.lax.fori_loop(0, _K, sel_body, (cmax, cbase0))

    cand = cand_ref[...]

    def thr_body(j, st):
        vals, t, cnt, cprev = st
        active = cnt < _K
        mx = jnp.max(vals)
        mult = jnp.sum((vals == mx).astype(jnp.int32))
        t = jnp.where(active, mx, t)
        cprev = jnp.where(active, cnt, cprev)
        cnt = jnp.where(active, cnt + mult, cnt)
        vals = jnp.where(jnp.logical_and(active, vals == mx), _NEG, vals)
        return vals, t, cnt, cprev

    _, t, _, c_above = jax.lax.fori_loop(
        0, _K, thr_body, (cand, _NEG, jnp.int32(0), jnp.int32(0)))
    e = _K - c_above

    lane_iota = jax.lax.broadcasted_iota(jnp.int32, (_K, _W), 1)
    flat_idx = cbase * _W + lane_iota
    fm0 = jnp.where(cand == t, flat_idx, _BIG)

    def tie_body(j, st):
        fm, last = st
        active = j < e
        cur = jnp.min(fm)
        last = jnp.where(active, cur, last)
        fm = jnp.where(jnp.logical_and(active, fm == cur), _BIG, fm)
        return fm, last

    _, i_e = jax.lax.fori_loop(0, _K, tie_body, (fm0, _BIG))

    full_iota = (jax.lax.broadcasted_iota(jnp.int32, (C, _W), 0) * _W
                 + jax.lax.broadcasted_iota(jnp.int32, (C, _W), 1))
    keep = jnp.logical_or(x > t,
                          jnp.logical_and(x == t, full_iota <= i_e))
    o_ref[0] = keep.astype(jnp.float32)


def kernel(logits):
    B, N = logits.shape
    C = N // _W
    x3 = logits.reshape(B, C, _W)
    out = pl.pallas_call(
        _row_kernel,
        grid=(B,),
        in_specs=[pl.BlockSpec((1, C, _W), lambda i: (i, 0, 0))],
        out_specs=pl.BlockSpec((1, C, _W), lambda i: (i, 0, 0)),
        out_shape=jax.ShapeDtypeStruct((B, C, _W), jnp.float32),
        scratch_shapes=[pltpu.VMEM((_K, _W), jnp.float32)],
        compiler_params=pltpu.CompilerParams(
            dimension_semantics=("arbitrary",),
        ),
    )(x3)
    return out.reshape(B, N)


# speedup vs baseline: 8.1367x; 2.3565x over previous
import jax
import jax.numpy as jnp
from jax.experimental import pallas as pl
from jax.experimental.pallas import tpu as pltpu

_K = 64
_W = 250
_RW = 2


def _mono(v):
    u = jax.lax.bitcast_convert_type(v, jnp.int32)
    return u ^ (jax.lax.shift_right_arithmetic(u, 31) & jnp.int32(0x7FFFFFFF))


def _rows_kernel(x_ref, o_ref, cand_ref):
    _BIG = jnp.int32(2**30)
    _INT_MIN = jnp.int32(-(2**31))
    x = x_ref[...]
    RW, C, W = x.shape
    mr = 125 if C % 125 == 0 else 128
    R = C // mr

    def cnt(pred):
        return jnp.sum(pred.astype(jnp.int32), axis=(1, 2), keepdims=True)

    ci = _mono(jnp.max(x, axis=2)).reshape(RW, R, mr)

    zero3 = jnp.zeros((RW, 1, 1), jnp.int32)
    tc = jnp.where(cnt(ci >= 0) >= _K, zero3, zero3 + _INT_MIN)

    def tc_body(b, t):
        t_try = t + jax.lax.shift_left(jnp.int32(1), jnp.int32(30) - b)
        return jnp.where(cnt(ci >= t_try) >= _K, t_try, t)

    tc = jax.lax.fori_loop(0, 31, tc_body, tc)
    s_sel = cnt(ci >= tc)

    _OFF = jnp.int32(8192)
    chunk_iota = (jax.lax.broadcasted_iota(jnp.int32, (RW, R, mr), 1) * mr
                  + jax.lax.broadcasted_iota(jnp.int32, (RW, R, mr), 2))
    mi0 = jnp.where(ci > tc, chunk_iota,
                    jnp.where(ci == tc, chunk_iota + _OFF, _BIG))
    row64 = jax.lax.broadcasted_iota(jnp.int32, (_K, 1), 0)
    cb0_init = jnp.zeros((_K, 1), jnp.int32)

    def g_body(k, st):
        mi, cb0, cb1 = st
        pv = jnp.min(mi, axis=(1, 2), keepdims=True)
        p0 = pv[0, 0, 0] & jnp.int32(8191)
        p1 = pv[1, 0, 0] & jnp.int32(8191)
        cand_ref[0, pl.ds(k, 1), :] = x_ref[0, pl.ds(p0, 1), :]
        cand_ref[1, pl.ds(k, 1), :] = x_ref[1, pl.ds(p1, 1), :]
        cb0 = jnp.where(row64 == k, p0, cb0)
        cb1 = jnp.where(row64 == k, p1, cb1)
        mi = jnp.where(mi == pv, _BIG, mi)
        return mi, cb0, cb1

    _, cb0, cb1 = jax.lax.fori_loop(0, _K, g_body, (mi0, cb0_init, cb0_init))

    candi = _mono(cand_ref[...])

    def t_body(b, t):
        t_try = t + jax.lax.shift_left(jnp.int32(1), jnp.int32(30) - b)
        return jnp.where(cnt(candi >= t_try) >= _K, t_try, t)

    tt = jnp.where(cnt(candi >= 0) >= _K, zero3, zero3 + _INT_MIN)
    tt = jax.lax.fori_loop(0, 31, t_body, tt)

    c_above = cnt(candi > tt)
    cnt_eq = cnt(candi == tt)
    e = _K - c_above
    t_f = jax.lax.bitcast_convert_type(
        tt ^ (jax.lax.shift_right_arithmetic(tt, 31) & jnp.int32(0x7FFFFFFF)),
        jnp.float32)

    fast = jnp.logical_and(
        cnt_eq == e,
        jnp.logical_or(tt > tc, s_sel == _K))
    fast_all = jnp.all(fast)

    @pl.when(fast_all)
    def _fast():
        o_ref[...] = (x >= t_f).astype(jnp.float32)

    @pl.when(jnp.logical_not(fast_all))
    def _slow():
        lane = jax.lax.broadcasted_iota(jnp.int32, (RW, _K, W), 2)
        cbs = jnp.stack([cb0, cb1])
        flat = cbs * W + lane
        eq = candi == tt

        def i_body(b, lo):
            add = jax.lax.shift_left(jnp.int32(1), jnp.int32(19) - b)
            i_mid = lo + add - 1
            c = cnt(jnp.logical_and(eq, flat <= i_mid))
            return jnp.where(c >= e, lo, lo + add)

        i_e = jax.lax.fori_loop(0, 20, i_body, zero3)
        full_iota = (jax.lax.broadcasted_iota(jnp.int32, (RW, C, W), 1) * W
                     + jax.lax.broadcasted_iota(jnp.int32, (RW, C, W), 2))
        keep = jnp.logical_or(
            x > t_f, jnp.logical_and(x == t_f, full_iota <= i_e))
        o_ref[...] = keep.astype(jnp.float32)


def kernel(logits):
    B, N = logits.shape
    C = N // _W
    x3 = logits.reshape(B, C, _W)
    out = pl.pallas_call(
        _rows_kernel,
        grid=(B // _RW,),
        in_specs=[pl.BlockSpec((_RW, C, _W), lambda i: (i, 0, 0))],
        out_specs=pl.BlockSpec((_RW, C, _W), lambda i: (i, 0, 0)),
        out_shape=jax.ShapeDtypeStruct((B, C, _W), jnp.float32),
        scratch_shapes=[pltpu.VMEM((_RW, _K, _W), jnp.float32)],
        compiler_params=pltpu.CompilerParams(
            dimension_semantics=("arbitrary",),
        ),
    )(x3)
    return out.reshape(B, N)
